# Initial kernel scaffold; baseline (speedup 1.0000x reference)
import jax
import jax.numpy as jnp
from jax.experimental import pallas as pl


def kernel(x, y):
    raise NotImplementedError("write your pallas kernel here")


# speedup vs baseline: 67.6909x; 67.6909x over previous
import jax
import jax.numpy as jnp
from jax import lax
from jax.experimental import pallas as pl
from jax.experimental.pallas import tpu as pltpu

_THR = 0.01
_NP_RATIO = 3.0
_HE_RATIO = 0.005

_SUB = 1152
_LANE = 128
_HW = _SUB * _LANE
_ROWS = 24


def _ohem_body(x_ref, y_ref, out_ref, acc_ref, bits_ref):
    r = pl.program_id(0)

    d = x_ref[0] - y_ref[0]
    loss = d * d
    neg = loss >= _THR
    bits = jnp.where(neg, lax.bitcast_convert_type(loss, jnp.int32), 0)
    bits_ref[...] = bits

    nneg = jnp.sum(neg.astype(jnp.int32))
    npos = _HW - nneg
    nneg_f = nneg.astype(jnp.float32)
    npos_f = npos.astype(jnp.float32)
    nhe = jnp.where(nneg_f > _NP_RATIO * npos_f, _NP_RATIO * npos_f, nneg_f)
    nhe = jnp.maximum(nhe, jnp.float32(_HE_RATIO * float(_HW)))
    k = jnp.floor(nhe).astype(jnp.int32)
    k_eff = jnp.minimum(k, nneg)

    def step(_, carry):
        lo, hi = carry
        mid = lo + ((hi - lo) >> 1)
        cnt = jnp.sum((bits_ref[...] >= mid).astype(jnp.int32))
        ge = cnt >= k_eff
        return jnp.where(ge, mid, lo), jnp.where(ge, hi, mid)

    lo, _ = lax.fori_loop(
        0, 31, step, (jnp.int32(0), jnp.int32(0x7F800000)), unroll=False
    )

    b = bits_ref[...]
    gt = b > lo
    c_gt = jnp.sum(gt.astype(jnp.int32))
    s_gt = jnp.sum(jnp.where(gt, lax.bitcast_convert_type(b, jnp.float32), 0.0))
    tval = lax.bitcast_convert_type(lo, jnp.float32)
    s_top = s_gt + (k_eff - c_gt).astype(jnp.float32) * tval
    l_row = jnp.where(nneg > 0, s_top / nhe, 0.0)

    @pl.when(r == 0)
    def _():
        acc_ref[0, 0] = 0.0

    acc_ref[0, 0] += l_row

    @pl.when(r == _ROWS - 1)
    def _():
        out_ref[0, 0] = acc_ref[0, 0] / jnp.float32(_ROWS)


def kernel(x, y):
    x2 = x.reshape(_ROWS, _SUB, _LANE)
    y2 = y.reshape(_ROWS, _SUB, _LANE)
    out = pl.pallas_call(
        _ohem_body,
        grid=(_ROWS,),
        in_specs=[
            pl.BlockSpec((1, _SUB, _LANE), lambda r: (r, 0, 0)),
            pl.BlockSpec((1, _SUB, _LANE), lambda r: (r, 0, 0)),
        ],
        out_specs=pl.BlockSpec(memory_space=pltpu.SMEM),
        out_shape=jax.ShapeDtypeStruct((1, 1), jnp.float32),
        scratch_shapes=[
            pltpu.SMEM((1, 1), jnp.float32),
            pltpu.VMEM((_SUB, _LANE), jnp.int32),
        ],
        compiler_params=pltpu.CompilerParams(
            dimension_semantics=("arbitrary",),
        ),
    )(x2, y2)
    return out[0, 0]


# speedup vs baseline: 165.6671x; 2.4474x over previous
import jax
import jax.numpy as jnp
from jax import lax
from jax.experimental import pallas as pl
from jax.experimental.pallas import tpu as pltpu

_THR = 0.01
_NP_RATIO = 3.0
_HE_RATIO = 0.005

_SUB = 1152
_LANE = 128
_HW = _SUB * _LANE
_ROWS = 24
_G = 8
_STEPS = _ROWS // _G


def _ohem_body(x_ref, y_ref, out_ref, acc_ref, bits_ref):
    r = pl.program_id(0)

    d = x_ref[0] - y_ref[0]
    loss = d * d
    neg = loss >= _THR
    bits = jnp.where(neg, lax.bitcast_convert_type(loss, jnp.int32), 0)
    bits_ref[...] = bits

    nneg = jnp.sum(neg.astype(jnp.int32), axis=(1, 2))
    npos = _HW - nneg
    nneg_f = nneg.astype(jnp.float32)
    npos_f = npos.astype(jnp.float32)
    nhe = jnp.where(nneg_f > _NP_RATIO * npos_f, _NP_RATIO * npos_f, nneg_f)
    nhe = jnp.maximum(nhe, jnp.float32(_HE_RATIO * float(_HW)))
    k = jnp.floor(nhe).astype(jnp.int32)
    k_eff = jnp.minimum(k, nneg)

    def step(_, carry):
        lo, hi = carry
        mid = lo + ((hi - lo) >> 1)
        cnt = jnp.sum(
            (bits_ref[...] >= mid[:, None, None]).astype(jnp.int32), axis=(1, 2)
        )
        ge = cnt >= k_eff
        return jnp.where(ge, mid, lo), jnp.where(ge, hi, mid)

    lo, _ = lax.fori_loop(
        0,
        31,
        step,
        (jnp.zeros((_G,), jnp.int32), jnp.full((_G,), 0x7F800000, jnp.int32)),
        unroll=False,
    )

    b = bits_ref[...]
    gt = b > lo[:, None, None]
    c_gt = jnp.sum(gt.astype(jnp.int32), axis=(1, 2))
    s_gt = jnp.sum(
        jnp.where(gt, lax.bitcast_convert_type(b, jnp.float32), 0.0), axis=(1, 2)
    )
    tval = lax.bitcast_convert_type(lo, jnp.float32)
    s_top = s_gt + (k_eff - c_gt).astype(jnp.float32) * tval
    l_rows = jnp.where(nneg > 0, s_top / nhe, 0.0)
    l_sum = jnp.sum(l_rows)

    @pl.when(r == 0)
    def _():
        acc_ref[0, 0] = 0.0

    acc_ref[0, 0] += l_sum

    @pl.when(r == _STEPS - 1)
    def _():
        out_ref[0, 0] = acc_ref[0, 0] / jnp.float32(_ROWS)


def kernel(x, y):
    x2 = x.reshape(_STEPS, _G, _SUB, _LANE)
    y2 = y.reshape(_STEPS, _G, _SUB, _LANE)
    out = pl.pallas_call(
        _ohem_body,
        grid=(_STEPS,),
        in_specs=[
            pl.BlockSpec((1, _G, _SUB, _LANE), lambda r: (r, 0, 0, 0)),
            pl.BlockSpec((1, _G, _SUB, _LANE), lambda r: (r, 0, 0, 0)),
        ],
        out_specs=pl.BlockSpec(memory_space=pltpu.SMEM),
        out_shape=jax.ShapeDtypeStruct((1, 1), jnp.float32),
        scratch_shapes=[
            pltpu.SMEM((1, 1), jnp.float32),
            pltpu.VMEM((_G, _SUB, _LANE), jnp.int32),
        ],
        compiler_params=pltpu.CompilerParams(
            dimension_semantics=("arbitrary",),
        ),
    )(x2, y2)
    return out[0, 0]


# speedup vs baseline: 175.1142x; 1.0570x over previous
import jax
import jax.numpy as jnp
from jax import lax
from jax.experimental import pallas as pl
from jax.experimental.pallas import tpu as pltpu

_THR = 0.01
_NP_RATIO = 3.0
_HE_RATIO = 0.005

_SUB = 1152
_LANE = 128
_HW = _SUB * _LANE
_ROWS = 24
_G = 24
_STEPS = _ROWS // _G


def _ohem_body(x_ref, y_ref, out_ref, acc_ref, bits_ref):
    r = pl.program_id(0)

    d = x_ref[0] - y_ref[0]
    loss = d * d
    neg = loss >= _THR
    bits = jnp.where(neg, lax.bitcast_convert_type(loss, jnp.int32), 0)
    bits_ref[...] = bits

    nneg = jnp.sum(neg.astype(jnp.int32), axis=(1, 2))
    npos = _HW - nneg
    nneg_f = nneg.astype(jnp.float32)
    npos_f = npos.astype(jnp.float32)
    nhe = jnp.where(nneg_f > _NP_RATIO * npos_f, _NP_RATIO * npos_f, nneg_f)
    nhe = jnp.maximum(nhe, jnp.float32(_HE_RATIO * float(_HW)))
    k = jnp.floor(nhe).astype(jnp.int32)
    k_eff = jnp.minimum(k, nneg)

    def step(_, carry):
        lo, hi = carry
        mid = lo + ((hi - lo) >> 1)
        cnt = jnp.sum(
            (bits_ref[...] >= mid[:, None, None]).astype(jnp.int32), axis=(1, 2)
        )
        ge = cnt >= k_eff
        return jnp.where(ge, mid, lo), jnp.where(ge, hi, mid)

    lo, _ = lax.fori_loop(
        0,
        31,
        step,
        (jnp.zeros((_G,), jnp.int32), jnp.full((_G,), 0x7F800000, jnp.int32)),
        unroll=False,
    )

    b = bits_ref[...]
    gt = b > lo[:, None, None]
    c_gt = jnp.sum(gt.astype(jnp.int32), axis=(1, 2))
    s_gt = jnp.sum(
        jnp.where(gt, lax.bitcast_convert_type(b, jnp.float32), 0.0), axis=(1, 2)
    )
    tval = lax.bitcast_convert_type(lo, jnp.float32)
    s_top = s_gt + (k_eff - c_gt).astype(jnp.float32) * tval
    l_rows = jnp.where(nneg > 0, s_top / nhe, 0.0)
    l_sum = jnp.sum(l_rows)

    @pl.when(r == 0)
    def _():
        acc_ref[0, 0] = 0.0

    acc_ref[0, 0] += l_sum

    @pl.when(r == _STEPS - 1)
    def _():
        out_ref[0, 0] = acc_ref[0, 0] / jnp.float32(_ROWS)


def kernel(x, y):
    x2 = x.reshape(_STEPS, _G, _SUB, _LANE)
    y2 = y.reshape(_STEPS, _G, _SUB, _LANE)
    out = pl.pallas_call(
        _ohem_body,
        grid=(_STEPS,),
        in_specs=[
            pl.BlockSpec((1, _G, _SUB, _LANE), lambda r: (r, 0, 0, 0)),
            pl.BlockSpec((1, _G, _SUB, _LANE), lambda r: (r, 0, 0, 0)),
        ],
        out_specs=pl.BlockSpec(memory_space=pltpu.SMEM),
        out_shape=jax.ShapeDtypeStruct((1, 1), jnp.float32),
        scratch_shapes=[
            pltpu.SMEM((1, 1), jnp.float32),
            pltpu.VMEM((_G, _SUB, _LANE), jnp.int32),
        ],
        compiler_params=pltpu.CompilerParams(
            dimension_semantics=("arbitrary",),
        ),
    )(x2, y2)
    return out[0, 0]
